# baseline (speedup 1.0000x reference)
import functools

import jax
import jax.numpy as jnp
from jax import lax
from jax.experimental import pallas as pl
from jax.experimental.pallas import tpu as pltpu
from jax.experimental.pallas import tpu_sc as plsc

N_NODES = 10000
D = 128
R = 10240
E = 320000
EP = 327680
EROWS = EP // 128
NC, NS = 2, 16
NW = NC * NS
RPT = R // NS

IB = 16
CW = 64
IB2 = IB
CH0 = 156
CH1 = EROWS // NS - CH0


def _sc_agg_body(with_counts, x_hbm, src_hbm, dst_hbm, *refs):
    if with_counts:
        aggp_hbm, cntp_hbm = refs[0], refs[1]
        refs = refs[2:]
    else:
        aggp_hbm = refs[0]
        refs = refs[1:]
    src_i, dst_i, rows, ones_v, zcnt_v, agg_s, cnt_s, sem_i, sem_g, sem_s, sem_c = refs

    c = lax.axis_index("c")
    s = lax.axis_index("s")

    def _zrow(i, _):
        for j in range(8):
            rows[0, i, pl.ds(j * 16, 16)] = jnp.zeros((16,), jnp.float32)
        return 0
    lax.fori_loop(0, CW, _zrow, 0)

    def _zc(i, _):
        zcnt_v[pl.ds(i * 16, 16)] = jnp.zeros((16,), jnp.float32)
        return 0
    lax.fori_loop(0, RPT // 16, _zc, 0)
    for j in range(CW // 16):
        ones_v[pl.ds(j * 16, 16)] = jnp.ones((16,), jnp.float32)

    for q in range(RPT // CW):
        pltpu.sync_copy(rows.at[0], agg_s.at[pl.ds(s * RPT + q * CW, CW)])
    if with_counts:
        pltpu.sync_copy(zcnt_v, cnt_s.at[pl.ds(s * RPT, RPT)])
    plsc.subcore_barrier()

    row0 = 2 * jnp.where(c == 0, s * CH0, NS * CH0 + s * CH1)
    nch = 2 * jnp.where(c == 0, CH0, CH1)
    nib = (nch + IB2 - 1) // IB2

    @pl.when(nch > 0)
    def _():
        pltpu.sync_copy(src_hbm.at[pl.ds(row0, IB2)], src_i.at[0])
        pltpu.sync_copy(dst_hbm.at[pl.ds(row0, IB2)], dst_i.at[0])

    @pl.when(nib > 1)
    def _():
        pltpu.async_copy(src_hbm.at[pl.ds(row0 + IB2, IB2)], src_i.at[1], sem_i)
        pltpu.async_copy(dst_hbm.at[pl.ds(row0 + IB2, IB2)], dst_i.at[1], sem_i)

    @pl.when(nch > 0)
    def _():
        pltpu.async_copy(x_hbm.at[src_i.at[0, 0]], rows.at[0], sem_g)
        pltpu.async_copy(x_hbm.at[src_i.at[0, 1]], rows.at[1], sem_g)

    def _wait(dst_ref, sem):
        pltpu.make_async_copy(x_hbm.at[pl.ds(0, dst_ref.shape[0])], dst_ref,
                              sem).wait()

    def _step(i, _):
        p = i % 4
        b = i // IB2
        r = i % IB2

        @pl.when(i >= 2)
        def _():
            _wait(rows.at[(i + 2) % 4], sem_s)
            if with_counts:
                pltpu.make_async_copy(x_hbm.at[0, pl.ds(0, CW)], ones_v,
                                      sem_c).wait()

        @pl.when(jnp.logical_and(r == 0,
                                 jnp.logical_and(b >= 1, b < nib - 1)))
        def _():
            nb = b + 1
            pltpu.async_copy(src_hbm.at[pl.ds(row0 + nb * IB2, IB2)],
                             src_i.at[nb % 2], sem_i)
            pltpu.async_copy(dst_hbm.at[pl.ds(row0 + nb * IB2, IB2)],
                             dst_i.at[nb % 2], sem_i)

        @pl.when(jnp.logical_and(r == IB2 - 2, i < nch - 2))
        def _():
            pltpu.make_async_copy(src_hbm.at[pl.ds(0, IB2)], src_i.at[0],
                                  sem_i).wait()
            pltpu.make_async_copy(dst_hbm.at[pl.ds(0, IB2)], dst_i.at[0],
                                  sem_i).wait()

        @pl.when(i < nch - 2)
        def _():
            n = i + 2
            pltpu.async_copy(x_hbm.at[src_i.at[(n // IB2) % 2, n % IB2]],
                             rows.at[n % 4], sem_g)

        _wait(rows.at[p], sem_g)
        pltpu.async_copy(rows.at[p], agg_s.at[dst_i.at[b % 2, r]],
                         sem_s, add=True)
        if with_counts:
            pltpu.async_copy(ones_v, cnt_s.at[dst_i.at[b % 2, r]],
                             sem_c, add=True)
        return 0
    lax.fori_loop(0, nch, _step, 0)

    @pl.when(nch > 0)
    def _():
        _wait(rows.at[(nch - 2) % 4], sem_s)
        _wait(rows.at[(nch - 1) % 4], sem_s)
        if with_counts:
            pltpu.make_async_copy(x_hbm.at[0, pl.ds(0, CW)], ones_v,
                                  sem_c).wait()
            pltpu.make_async_copy(x_hbm.at[0, pl.ds(0, CW)], ones_v,
                                  sem_c).wait()
    plsc.subcore_barrier()

    for q in range(RPT // 128):
        off = s * RPT + q * 128
        pltpu.sync_copy(agg_s.at[pl.ds(off, 128)],
                        aggp_hbm.at[c, pl.ds(off, 128)])
    if with_counts:
        pltpu.sync_copy(cnt_s.at[pl.ds(s * RPT, RPT)],
                        cntp_hbm.at[c, pl.ds(s * RPT, RPT)])


@functools.lru_cache(maxsize=None)
def _make_sc_agg(with_counts):
    if with_counts:
        out_type = [jax.ShapeDtypeStruct((NC, R, D), jnp.float32),
                    jax.ShapeDtypeStruct((NC, R), jnp.float32)]
    else:
        out_type = jax.ShapeDtypeStruct((NC, R, D), jnp.float32)
    return pl.kernel(
        functools.partial(_sc_agg_body, with_counts),
        out_type=out_type,
        mesh=plsc.VectorSubcoreMesh(core_axis_name="c", subcore_axis_name="s",
                                    num_cores=NC, num_subcores=NS),
        scratch_types=[
            pltpu.VMEM((2, IB2, CW), jnp.int32),
            pltpu.VMEM((2, IB2, CW), jnp.int32),
            pltpu.VMEM((4, CW, D), jnp.float32),
            pltpu.VMEM((CW,), jnp.float32),
            pltpu.VMEM((RPT,), jnp.float32),
            pltpu.VMEM_SHARED((R, D), jnp.float32),
            pltpu.VMEM_SHARED((R,), jnp.float32),
            pltpu.SemaphoreType.DMA,
            pltpu.SemaphoreType.DMA,
            pltpu.SemaphoreType.DMA,
            pltpu.SemaphoreType.DMA,
        ],
    )


def _dense_body(aggp_ref, cnt_ref, x_ref, wl_ref, wr_ref, bl_ref, out_ref):
    agg = aggp_ref[0] + aggp_ref[1]
    cnt = cnt_ref[0] + cnt_ref[1]
    inv = 1.0 / jnp.maximum(cnt, 1.0)
    h = lax.dot_general(agg * inv[:, None], wl_ref[...],
                        (((1,), (1,)), ((), ())),
                        preferred_element_type=jnp.float32)
    h = h + lax.dot_general(x_ref[...], wr_ref[...],
                            (((1,), (1,)), ((), ())),
                            preferred_element_type=jnp.float32)
    out_ref[...] = jnp.maximum(h + bl_ref[...], 0.0)


def _dense_final_body(aggp_ref, cnt_ref, x_ref, wl_ref, wr_ref, bl_ref,
                      wlin_ref, blin_ref, out_ref):
    agg = aggp_ref[0] + aggp_ref[1]
    cnt = cnt_ref[0] + cnt_ref[1]
    inv = 1.0 / jnp.maximum(cnt, 1.0)
    h = lax.dot_general(agg * inv[:, None], wl_ref[...],
                        (((1,), (1,)), ((), ())),
                        preferred_element_type=jnp.float32)
    h = h + lax.dot_general(x_ref[...], wr_ref[...],
                            (((1,), (1,)), ((), ())),
                            preferred_element_type=jnp.float32)
    h = jnp.maximum(h + bl_ref[...], 0.0)
    out_ref[...] = lax.dot_general(h, wlin_ref[...],
                                   (((1,), (1,)), ((), ())),
                                   preferred_element_type=jnp.float32) + blin_ref[...]


_BR = 10240
_GRID = R // _BR

_aggp_spec = pl.BlockSpec((NC, _BR, D), lambda i: (0, i, 0))
_cnt_spec = pl.BlockSpec((NC, _BR), lambda i: (0, i))
_x_spec = pl.BlockSpec((_BR, D), lambda i: (i, 0))
_w_spec = pl.BlockSpec((D, D), lambda i: (0, 0))
_b_spec = pl.BlockSpec((1, D), lambda i: (0, 0))
_out_spec = pl.BlockSpec((_BR, D), lambda i: (i, 0))

_dense = pl.pallas_call(
    _dense_body,
    grid=(_GRID,),
    in_specs=[_aggp_spec, _cnt_spec, _x_spec, _w_spec, _w_spec, _b_spec],
    out_specs=_out_spec,
    out_shape=jax.ShapeDtypeStruct((R, D), jnp.float32),
)

_dense_final = pl.pallas_call(
    _dense_final_body,
    grid=(_GRID,),
    in_specs=[_aggp_spec, _cnt_spec, _x_spec, _w_spec, _w_spec, _b_spec,
              _w_spec, _b_spec],
    out_specs=_out_spec,
    out_shape=jax.ShapeDtypeStruct((R, D), jnp.float32),
)


def kernel(x, edge_index, Wl1, bl1, Wr1, Wl2, bl2, Wr2, Wlin, blin):
    src = edge_index[0].astype(jnp.int32)
    dst = edge_index[1].astype(jnp.int32)
    pad = EP + IB * 128 - E
    src_p = jnp.concatenate([src, jnp.zeros((pad,), jnp.int32)]).reshape(2 * (EROWS + IB), CW)
    dst_p = jnp.concatenate([dst, jnp.full((pad,), N_NODES, jnp.int32)]).reshape(2 * (EROWS + IB), CW)
    x_p = jnp.concatenate([x, jnp.zeros((R - N_NODES, D), jnp.float32)])

    bl1_2 = bl1.reshape(1, D)
    bl2_2 = bl2.reshape(1, D)
    blin_2 = blin.reshape(1, D)

    aggp1, cntp = _make_sc_agg(True)(x_p, src_p, dst_p)
    h1 = _dense(aggp1, cntp, x_p, Wl1, Wr1, bl1_2)
    aggp2 = _make_sc_agg(False)(h1, src_p, dst_p)
    out_p = _dense_final(aggp2, cntp, h1, Wl2, Wr2, bl2_2, Wlin, blin_2)
    return out_p[:N_NODES]


# speedup vs baseline: 1.0042x; 1.0042x over previous
import functools

import jax
import jax.numpy as jnp
from jax import lax
from jax.experimental import pallas as pl
from jax.experimental.pallas import tpu as pltpu
from jax.experimental.pallas import tpu_sc as plsc

N_NODES = 10000
D = 128
R = 10240
E = 320000
EP = 327680
EROWS = EP // 128
NC, NS = 2, 16
NW = NC * NS
RPT = R // NS

IB = 16
CW = 64
IB2 = IB
CH0 = 156
CH1 = EROWS // NS - CH0


def _sc_agg_body(with_counts, x_hbm, src_hbm, dst_hbm, *refs):
    if with_counts:
        aggp_hbm, cntp_hbm = refs[0], refs[1]
        refs = refs[2:]
    else:
        aggp_hbm = refs[0]
        refs = refs[1:]
    src_i, dst_i, rows, ones_v, zcnt_v, agg_s, cnt_s, sem_i, sem_g, sem_s, sem_c = refs

    c = lax.axis_index("c")
    s = lax.axis_index("s")

    def _zrow(i, _):
        for j in range(8):
            rows[0, i, pl.ds(j * 16, 16)] = jnp.zeros((16,), jnp.float32)
        return 0
    lax.fori_loop(0, CW, _zrow, 0)

    def _zc(i, _):
        zcnt_v[pl.ds(i * 16, 16)] = jnp.zeros((16,), jnp.float32)
        return 0
    lax.fori_loop(0, RPT // 16, _zc, 0)
    for j in range(CW // 16):
        ones_v[pl.ds(j * 16, 16)] = jnp.ones((16,), jnp.float32)

    for q in range(RPT // CW):
        pltpu.sync_copy(rows.at[0], agg_s.at[pl.ds(s * RPT + q * CW, CW)])
    if with_counts:
        pltpu.sync_copy(zcnt_v, cnt_s.at[pl.ds(s * RPT, RPT)])
    plsc.subcore_barrier()

    row0 = 2 * jnp.where(c == 0, s * CH0, NS * CH0 + s * CH1)
    nch = 2 * jnp.where(c == 0, CH0, CH1)
    nib = (nch + IB2 - 1) // IB2

    @pl.when(nch > 0)
    def _():
        pltpu.sync_copy(src_hbm.at[pl.ds(row0, IB2)], src_i.at[0])
        pltpu.sync_copy(dst_hbm.at[pl.ds(row0, IB2)], dst_i.at[0])

    @pl.when(nib > 1)
    def _():
        pltpu.async_copy(src_hbm.at[pl.ds(row0 + IB2, IB2)], src_i.at[1], sem_i)
        pltpu.async_copy(dst_hbm.at[pl.ds(row0 + IB2, IB2)], dst_i.at[1], sem_i)

    @pl.when(nch > 0)
    def _():
        pltpu.async_copy(x_hbm.at[src_i.at[0, 0]], rows.at[0], sem_g)
        pltpu.async_copy(x_hbm.at[src_i.at[0, 1]], rows.at[1], sem_g)

    def _wait(dst_ref, sem):
        pltpu.make_async_copy(x_hbm.at[pl.ds(0, dst_ref.shape[0])], dst_ref,
                              sem).wait()

    def _step(i, _):
        p = i % 4
        b = i // IB2
        r = i % IB2

        @pl.when(i >= 2)
        def _():
            _wait(rows.at[(i + 2) % 4], sem_s)
            if with_counts:
                pltpu.make_async_copy(x_hbm.at[0, pl.ds(0, CW)], ones_v,
                                      sem_c).wait()

        @pl.when(jnp.logical_and(r == 0,
                                 jnp.logical_and(b >= 1, b < nib - 1)))
        def _():
            nb = b + 1
            pltpu.async_copy(src_hbm.at[pl.ds(row0 + nb * IB2, IB2)],
                             src_i.at[nb % 2], sem_i)
            pltpu.async_copy(dst_hbm.at[pl.ds(row0 + nb * IB2, IB2)],
                             dst_i.at[nb % 2], sem_i)

        @pl.when(jnp.logical_and(r == IB2 - 2, i < nch - 2))
        def _():
            pltpu.make_async_copy(src_hbm.at[pl.ds(0, IB2)], src_i.at[0],
                                  sem_i).wait()
            pltpu.make_async_copy(dst_hbm.at[pl.ds(0, IB2)], dst_i.at[0],
                                  sem_i).wait()

        @pl.when(i < nch - 2)
        def _():
            n = i + 2
            pltpu.async_copy(x_hbm.at[src_i.at[(n // IB2) % 2, n % IB2]],
                             rows.at[n % 4], sem_g)

        _wait(rows.at[p], sem_g)
        pltpu.async_copy(rows.at[p], agg_s.at[dst_i.at[b % 2, r]],
                         sem_s, add=True)
        if with_counts:
            pltpu.async_copy(ones_v, cnt_s.at[dst_i.at[b % 2, r]],
                             sem_c, add=True)
        return 0
    lax.fori_loop(0, nch, _step, 0)

    @pl.when(nch > 0)
    def _():
        _wait(rows.at[(nch - 2) % 4], sem_s)
        _wait(rows.at[(nch - 1) % 4], sem_s)
        if with_counts:
            pltpu.make_async_copy(x_hbm.at[0, pl.ds(0, CW)], ones_v,
                                  sem_c).wait()
            pltpu.make_async_copy(x_hbm.at[0, pl.ds(0, CW)], ones_v,
                                  sem_c).wait()
    plsc.subcore_barrier()

    for q in range(RPT // 128):
        off = s * RPT + q * 128
        pltpu.sync_copy(agg_s.at[pl.ds(off, 128)],
                        aggp_hbm.at[c, pl.ds(off, 128)])
    if with_counts:
        pltpu.sync_copy(cnt_s.at[pl.ds(s * RPT, RPT)],
                        cntp_hbm.at[c, pl.ds(s * RPT, RPT)])


@functools.lru_cache(maxsize=None)
def _make_sc_agg(with_counts):
    if with_counts:
        out_type = [jax.ShapeDtypeStruct((NC, R, D), jnp.float32),
                    jax.ShapeDtypeStruct((NC, R), jnp.float32)]
    else:
        out_type = jax.ShapeDtypeStruct((NC, R, D), jnp.float32)
    return pl.kernel(
        functools.partial(_sc_agg_body, with_counts),
        out_type=out_type,
        mesh=plsc.VectorSubcoreMesh(core_axis_name="c", subcore_axis_name="s",
                                    num_cores=NC, num_subcores=NS),
        scratch_types=[
            pltpu.VMEM((2, IB2, CW), jnp.int32),
            pltpu.VMEM((2, IB2, CW), jnp.int32),
            pltpu.VMEM((4, CW, D), jnp.float32),
            pltpu.VMEM((CW,), jnp.float32),
            pltpu.VMEM((RPT,), jnp.float32),
            pltpu.VMEM_SHARED((R, D), jnp.float32),
            pltpu.VMEM_SHARED((R,), jnp.float32),
            pltpu.SemaphoreType.DMA,
            pltpu.SemaphoreType.DMA,
            pltpu.SemaphoreType.DMA,
            pltpu.SemaphoreType.DMA,
        ],
    )


def _dense_body(aggp_ref, cnt_ref, x_ref, wl_ref, wr_ref, bl_ref, out_ref):
    agg = aggp_ref[0] + aggp_ref[1]
    cnt = cnt_ref[0] + cnt_ref[1]
    inv = 1.0 / jnp.maximum(cnt, 1.0)
    h = lax.dot_general(agg * inv[:, None], wl_ref[...],
                        (((1,), (1,)), ((), ())),
                        preferred_element_type=jnp.float32)
    h = h + lax.dot_general(x_ref[...], wr_ref[...],
                            (((1,), (1,)), ((), ())),
                            preferred_element_type=jnp.float32)
    out_ref[...] = jnp.maximum(h + bl_ref[...], 0.0)


def _dense_final_body(aggp_ref, cnt_ref, x_ref, wl_ref, wr_ref, bl_ref,
                      wlin_ref, blin_ref, out_ref):
    agg = aggp_ref[0] + aggp_ref[1]
    cnt = cnt_ref[0] + cnt_ref[1]
    inv = 1.0 / jnp.maximum(cnt, 1.0)
    h = lax.dot_general(agg * inv[:, None], wl_ref[...],
                        (((1,), (1,)), ((), ())),
                        preferred_element_type=jnp.float32)
    h = h + lax.dot_general(x_ref[...], wr_ref[...],
                            (((1,), (1,)), ((), ())),
                            preferred_element_type=jnp.float32)
    h = jnp.maximum(h + bl_ref[...], 0.0)
    out_ref[...] = lax.dot_general(h, wlin_ref[...],
                                   (((1,), (1,)), ((), ())),
                                   preferred_element_type=jnp.float32) + blin_ref[...]


_BR = 5120
_GRID = R // _BR

_aggp_spec = pl.BlockSpec((NC, _BR, D), lambda i: (0, i, 0))
_cnt_spec = pl.BlockSpec((NC, _BR), lambda i: (0, i))
_x_spec = pl.BlockSpec((_BR, D), lambda i: (i, 0))
_w_spec = pl.BlockSpec((D, D), lambda i: (0, 0))
_b_spec = pl.BlockSpec((1, D), lambda i: (0, 0))
_out_spec = pl.BlockSpec((_BR, D), lambda i: (i, 0))

_dense = pl.pallas_call(
    _dense_body,
    grid=(_GRID,),
    in_specs=[_aggp_spec, _cnt_spec, _x_spec, _w_spec, _w_spec, _b_spec],
    out_specs=_out_spec,
    out_shape=jax.ShapeDtypeStruct((R, D), jnp.float32),
)

_dense_final = pl.pallas_call(
    _dense_final_body,
    grid=(_GRID,),
    in_specs=[_aggp_spec, _cnt_spec, _x_spec, _w_spec, _w_spec, _b_spec,
              _w_spec, _b_spec],
    out_specs=_out_spec,
    out_shape=jax.ShapeDtypeStruct((R, D), jnp.float32),
)


def kernel(x, edge_index, Wl1, bl1, Wr1, Wl2, bl2, Wr2, Wlin, blin):
    src = edge_index[0].astype(jnp.int32)
    dst = edge_index[1].astype(jnp.int32)
    pad = EP + IB * 128 - E
    src_p = jnp.concatenate([src, jnp.zeros((pad,), jnp.int32)]).reshape(2 * (EROWS + IB), CW)
    dst_p = jnp.concatenate([dst, jnp.full((pad,), N_NODES, jnp.int32)]).reshape(2 * (EROWS + IB), CW)
    x_p = jnp.concatenate([x, jnp.zeros((R - N_NODES, D), jnp.float32)])

    bl1_2 = bl1.reshape(1, D)
    bl2_2 = bl2.reshape(1, D)
    blin_2 = blin.reshape(1, D)

    aggp1, cntp = _make_sc_agg(True)(x_p, src_p, dst_p)
    h1 = _dense(aggp1, cntp, x_p, Wl1, Wr1, bl1_2)
    aggp2 = _make_sc_agg(False)(h1, src_p, dst_p)
    out_p = _dense_final(aggp2, cntp, h1, Wl2, Wr2, bl2_2, Wlin, blin_2)
    return out_p[:N_NODES]
